# baseline (speedup 1.0000x reference)
import functools

import jax
import jax.numpy as jnp
from jax import lax
from jax.experimental import pallas as pl
from jax.experimental.pallas import tpu as pltpu
from jax.experimental.pallas import tpu_sc as plsc

D = 16
NC = 2
NS = 16
NW = NC * NS
BATCH = 16384
PER_W = BATCH // NW
WAVE = 16
NWAVE = PER_W // WAVE

_mesh = plsc.VectorSubcoreMesh(core_axis_name="c", subcore_axis_name="s")


@functools.partial(
    pl.kernel,
    mesh=_mesh,
    compiler_params=pltpu.CompilerParams(
        needs_layout_passes=False,
        use_tc_tiling_on_sc=True,
        disable_bounds_checks=True,
    ),
    out_type=jax.ShapeDtypeStruct((BATCH,), jnp.float32),
    scratch_types=[
        pltpu.VMEM((PER_W,), jnp.int32),
        pltpu.VMEM((PER_W,), jnp.int32),
        pltpu.VMEM((WAVE * D, 128), jnp.float32),
        pltpu.VMEM((WAVE * D, 128), jnp.float32),
        pltpu.VMEM((PER_W,), jnp.float32),
        pltpu.VMEM((24, 128), jnp.float32),
        pltpu.SemaphoreType.DMA,
        pltpu.SemaphoreType.DMA,
    ],
)
def _gmf_sc(uidx_hbm, iidx_hbm, ut_hbm, it_hbm, wb_hbm, out_hbm,
            uiv, iiv, uslab, islab, obuf_v, wb_v, usem, isem):
    wid = lax.axis_index("s") * NC + lax.axis_index("c")
    base = wid * PER_W

    pltpu.sync_copy(uidx_hbm.at[pl.ds(base, PER_W)], uiv)
    pltpu.sync_copy(iidx_hbm.at[pl.ds(base, PER_W)], iiv)
    pltpu.sync_copy(wb_hbm, wb_v)

    iot = lax.iota(jnp.int32, D)
    wregs = [wb_v[d, pl.ds(0, D)] for d in range(D)]
    bias = wb_v[D, pl.ds(0, D)]

    def wave_body(w, carry):
        uqv = (uiv[pl.ds(w * WAVE, WAVE)] >> 7) << 7
        iqv = (iiv[pl.ds(w * WAVE, WAVE)] >> 7) << 7
        uhandles, ihandles = [], []
        for j in range(WAVE):
            uq = pl.multiple_of(uqv[j], 128)
            iq = pl.multiple_of(iqv[j], 128)
            uhandles.append(pltpu.async_copy(
                ut_hbm.at[:, pl.ds(uq, 128)],
                uslab.at[pl.ds(j * D, D)], usem))
            ihandles.append(pltpu.async_copy(
                it_hbm.at[:, pl.ds(iq, 128)],
                islab.at[pl.ds(j * D, D)], isem))
        for h in uhandles:
            h.wait()

        uc = uiv[pl.ds(w * WAVE, WAVE)] & 127
        ic = iiv[pl.ds(w * WAVE, WAVE)] & 127
        uvals = []
        for d in range(D):
            rows = iot * D + d
            uvals.append(plsc.load_gather(uslab, [rows, uc]) * wregs[d])
        for h in ihandles:
            h.wait()
        acc = bias
        for d in range(D):
            rows = iot * D + d
            acc = acc + uvals[d] * plsc.load_gather(islab, [rows, ic])
        obuf_v[pl.ds(w * WAVE, WAVE)] = jnp.maximum(acc, 0.0)
        return carry

    lax.fori_loop(0, NWAVE, wave_body, 0)

    pltpu.sync_copy(obuf_v, out_hbm.at[pl.ds(base, PER_W)])


def kernel(user, item, user_table, item_table, W, b):
    u = user.astype(jnp.int32)
    i = item.astype(jnp.int32)
    ut_t = user_table.T
    it_t = item_table.T
    wb = jnp.concatenate(
        [
            jnp.broadcast_to(W.reshape(D, 1), (D, 128)),
            jnp.broadcast_to(b.reshape(1, 1), (1, 128)),
            jnp.zeros((24 - D - 1, 128), jnp.float32),
        ],
        axis=0,
    )
    out = _gmf_sc(u, i, ut_t, it_t, wb)
    return out.reshape(BATCH, 1)


# speedup vs baseline: 1.0592x; 1.0592x over previous
import functools

import jax
import jax.numpy as jnp
from jax import lax
from jax.experimental import pallas as pl
from jax.experimental.pallas import tpu as pltpu
from jax.experimental.pallas import tpu_sc as plsc

D = 16
NC = 2
NS = 16
NW = NC * NS
BATCH = 16384
PER_W = BATCH // NW
WAVE = 16
NWAVE = PER_W // WAVE

_mesh = plsc.VectorSubcoreMesh(core_axis_name="c", subcore_axis_name="s")


@functools.partial(
    pl.kernel,
    mesh=_mesh,
    compiler_params=pltpu.CompilerParams(
        needs_layout_passes=False,
        use_tc_tiling_on_sc=True,
        disable_bounds_checks=True,
    ),
    out_type=jax.ShapeDtypeStruct((BATCH,), jnp.float32),
    scratch_types=[
        pltpu.VMEM((PER_W,), jnp.int32),
        pltpu.VMEM((PER_W,), jnp.int32),
        pltpu.VMEM((WAVE * D, 128), jnp.float32),
        pltpu.VMEM((WAVE * D, 128), jnp.float32),
        pltpu.VMEM((PER_W,), jnp.float32),
        pltpu.VMEM((24, 128), jnp.float32),
        pltpu.SemaphoreType.DMA,
    ],
)
def _gmf_sc(uidx_hbm, iidx_hbm, ut_hbm, it_hbm, wb_hbm, out_hbm,
            uiv, iiv, uslab, islab, obuf_v, wb_v, sem):
    wid = lax.axis_index("s") * NC + lax.axis_index("c")
    base = wid * PER_W

    pltpu.sync_copy(uidx_hbm.at[pl.ds(base, PER_W)], uiv)
    pltpu.sync_copy(iidx_hbm.at[pl.ds(base, PER_W)], iiv)
    pltpu.sync_copy(wb_hbm, wb_v)

    iot = lax.iota(jnp.int32, D)
    wregs = [wb_v[d, pl.ds(0, D)] for d in range(D)]
    bias = wb_v[D, pl.ds(0, D)]

    def wave_body(w, carry):
        uqv = (uiv[pl.ds(w * WAVE, WAVE)] >> 7) << 7
        iqv = (iiv[pl.ds(w * WAVE, WAVE)] >> 7) << 7
        handles = []
        for j in range(WAVE):
            uq = pl.multiple_of(uqv[j], 128)
            iq = pl.multiple_of(iqv[j], 128)
            handles.append(pltpu.async_copy(
                ut_hbm.at[:, pl.ds(uq, 128)],
                uslab.at[pl.ds(j * D, D)], sem))
            handles.append(pltpu.async_copy(
                it_hbm.at[:, pl.ds(iq, 128)],
                islab.at[pl.ds(j * D, D)], sem))
        for h in handles:
            h.wait()

        uc = uiv[pl.ds(w * WAVE, WAVE)] & 127
        ic = iiv[pl.ds(w * WAVE, WAVE)] & 127
        acc = bias
        for d in range(D):
            rows = iot * D + d
            uv = plsc.load_gather(uslab, [rows, uc])
            iv = plsc.load_gather(islab, [rows, ic])
            acc = acc + uv * iv * wregs[d]
        obuf_v[pl.ds(w * WAVE, WAVE)] = jnp.maximum(acc, 0.0)
        return carry

    lax.fori_loop(0, NWAVE, wave_body, 0)

    pltpu.sync_copy(obuf_v, out_hbm.at[pl.ds(base, PER_W)])


def kernel(user, item, user_table, item_table, W, b):
    u = user.astype(jnp.int32)
    i = item.astype(jnp.int32)
    ut_t = user_table.T
    it_t = item_table.T
    wb = jnp.concatenate(
        [
            jnp.broadcast_to(W.reshape(D, 1), (D, 128)),
            jnp.broadcast_to(b.reshape(1, 1), (1, 128)),
            jnp.zeros((24 - D - 1, 128), jnp.float32),
        ],
        axis=0,
    )
    out = _gmf_sc(u, i, ut_t, it_t, wb)
    return out.reshape(BATCH, 1)
